# baseline (speedup 1.0000x reference)
import functools

import jax
import jax.numpy as jnp
from jax import lax
from jax.experimental import pallas as pl
from jax.experimental.pallas import tpu as pltpu
from jax.experimental.pallas import tpu_sc as plsc

B = 16384
D = 64

NC = 2
NS = 16
NW = NC * NS
BPW = B // NW
CH = 128
NCH = BPW // CH


def _body(weapon, rank, sub_w, spec_w, W_weapon, W_rank, W_sub, W_special,
          e_w, e_r, e_s, e_sp, idx_v, rows_a, rows_b, gsem, osem):
  wid = lax.axis_index("s") * NC + lax.axis_index("c")
  base = wid * BPW

  pltpu.sync_copy(weapon.at[pl.ds(base, BPW)], idx_v.at[0])
  pltpu.sync_copy(rank.at[pl.ds(base, BPW)], idx_v.at[1])
  pltpu.sync_copy(sub_w.at[pl.ds(base, BPW)], idx_v.at[2])
  pltpu.sync_copy(spec_w.at[pl.ds(base, BPW)], idx_v.at[3])

  tables = (W_weapon, W_rank, W_sub, W_special)
  outs = (e_w, e_r, e_s, e_sp)
  bufs = (rows_a, rows_b)
  out_dma = None
  for t in range(4):
    buf = bufs[t % 2]
    copies = [
        pltpu.async_copy(
            tables[t].at[idx_v.at[t, pl.ds(j * CH, CH)]],
            buf.at[pl.ds(j * CH, CH), :], gsem)
        for j in range(NCH)
    ]
    for c in copies:
      c.wait()
    if out_dma is not None:
      out_dma.wait()
    out_dma = pltpu.async_copy(
        buf, outs[t].at[pl.ds(base, BPW), :], osem)
  out_dma.wait()


_OUT = jax.ShapeDtypeStruct((B, D), jnp.float32)

_embed = functools.partial(
    pl.kernel,
    out_type=[_OUT, _OUT, _OUT, _OUT],
    mesh=plsc.VectorSubcoreMesh(core_axis_name="c", subcore_axis_name="s"),
    compiler_params=pltpu.CompilerParams(use_tc_tiling_on_sc=False,
                                         needs_layout_passes=False),
    scratch_types=[
        pltpu.VMEM((4, BPW), jnp.int32),
        pltpu.VMEM((BPW, D), jnp.float32),
        pltpu.VMEM((BPW, D), jnp.float32),
        pltpu.SemaphoreType.DMA,
        pltpu.SemaphoreType.DMA,
    ],
)(_body)


def kernel(weapon, rank, level, sub_weapon, special_weapon, weapon_range,
           weapon_power, weapon_rounds_per, weapon_iine,
           W_weapon, W_rank, W_sub, W_special):
  e_w, e_r, e_s, e_sp = _embed(weapon, rank, sub_weapon, special_weapon,
                               W_weapon, W_rank, W_sub, W_special)
  return jnp.concatenate([
      e_w, e_r, e_s, e_sp, level[:, None], weapon_range[:, None],
      weapon_power[:, None], weapon_rounds_per[:, None],
      weapon_iine[:, None]], axis=1)


# speedup vs baseline: 1.3991x; 1.3991x over previous
import functools

import jax
import jax.numpy as jnp
from jax import lax
from jax.experimental import pallas as pl
from jax.experimental.pallas import tpu as pltpu
from jax.experimental.pallas import tpu_sc as plsc

B = 16384
D = 64
NFEAT = 5
OUT_W = 4 * D + NFEAT

NC = 2
NS = 16
L = 16
NW = NC * NS
BPW = B // NW
CH = 128
NCH = BPW // CH


def _fire_gathers(table, idx_v, t, buf, sem):
  return [
      pltpu.async_copy(
          table.at[idx_v.at[t, pl.ds(j * CH, CH)]],
          buf.at[pl.ds(j * CH, CH), :], sem)
      for j in range(NCH)
  ]


def _body(weapon, rank, sub_w, spec_w, level, wrange, wpower, wrounds,
          wiine, W_weapon, W_rank, W_sub, W_special, out,
          idx_v, rows_a, rows_b, feats_v, sbuf_v, isem, gsem, osem):
  wid = lax.axis_index("s") * NC + lax.axis_index("c")
  base = wid * BPW

  stage = []
  for i, ref in enumerate((weapon, rank, sub_w, spec_w)):
    stage.append(pltpu.async_copy(ref.at[pl.ds(base, BPW)], idx_v.at[i],
                                  isem))
  for f, ref in enumerate((level, wrange, wpower, wrounds, wiine)):
    stage.append(pltpu.async_copy(ref.at[pl.ds(base, BPW)], feats_v.at[f],
                                  isem))
  for c in stage:
    c.wait()

  tables = (W_weapon, W_rank, W_sub, W_special)
  bufs = (rows_a, rows_b)

  gathers = _fire_gathers(tables[0], idx_v, 0, bufs[0], gsem)
  out_dmas = [None, None]
  for t in range(4):
    nxt = None
    if t + 1 < 4:
      if out_dmas[(t + 1) % 2] is not None:
        out_dmas[(t + 1) % 2].wait()
        out_dmas[(t + 1) % 2] = None
      nxt = _fire_gathers(tables[t + 1], idx_v, t + 1, bufs[(t + 1) % 2],
                          gsem)
    if t == 0:
      for f in range(NFEAT):
        col = jnp.full((L,), f, jnp.int32)
        for j in range(BPW // L):
          vals = feats_v[f, pl.ds(j * L, L)]
          rows = lax.iota(jnp.int32, L) + (j * L)
          plsc.store_scatter(sbuf_v, [rows, col], vals)
    for c in gathers:
      c.wait()
    out_dmas[t % 2] = pltpu.async_copy(
        bufs[t % 2], out.at[pl.ds(base, BPW), pl.ds(t * D, D)], osem)
    gathers = nxt

  pltpu.sync_copy(sbuf_v, out.at[pl.ds(base, BPW), pl.ds(4 * D, NFEAT)])
  for d in out_dmas:
    if d is not None:
      d.wait()


_embed = functools.partial(
    pl.kernel,
    out_type=jax.ShapeDtypeStruct((B, OUT_W), jnp.float32),
    mesh=plsc.VectorSubcoreMesh(core_axis_name="c", subcore_axis_name="s"),
    compiler_params=pltpu.CompilerParams(use_tc_tiling_on_sc=False,
                                         needs_layout_passes=False),
    scratch_types=[
        pltpu.VMEM((4, BPW), jnp.int32),
        pltpu.VMEM((BPW, D), jnp.float32),
        pltpu.VMEM((BPW, D), jnp.float32),
        pltpu.VMEM((NFEAT, BPW), jnp.float32),
        pltpu.VMEM((BPW, NFEAT), jnp.float32),
        pltpu.SemaphoreType.DMA,
        pltpu.SemaphoreType.DMA,
        pltpu.SemaphoreType.DMA,
    ],
)(_body)


def kernel(weapon, rank, level, sub_weapon, special_weapon, weapon_range,
           weapon_power, weapon_rounds_per, weapon_iine,
           W_weapon, W_rank, W_sub, W_special):
  return _embed(weapon, rank, sub_weapon, special_weapon, level,
                weapon_range, weapon_power, weapon_rounds_per, weapon_iine,
                W_weapon, W_rank, W_sub, W_special)
